# baseline (speedup 1.0000x reference)
import jax
import jax.numpy as jnp
from jax.experimental import pallas as pl
from jax.experimental.pallas import tpu as pltpu

GRID_H, GRID_W, EMBED_DIM = 32, 32, 768
D = EMBED_DIM // 3
N = GRID_H * GRID_W


def _pos_emb_kernel(row_ref, col_ref, time_ref, cls_ref, out_ref,
                    row_s, col_s, sem_t, sem_o):
    ct = pltpu.make_async_copy(
        time_ref, out_ref.at[pl.ds(1, N), 0, pl.ds(2 * D, D)], sem_t)
    ct.start()

    row_s[...] = jnp.broadcast_to(
        row_ref[...][:, None, :], (GRID_H, GRID_W, D)).reshape(N, D)
    col_s[...] = jnp.broadcast_to(
        col_ref[...][None, :, :], (GRID_H, GRID_W, D)).reshape(N, D)

    cr = pltpu.make_async_copy(
        row_s, out_ref.at[pl.ds(1, N), 0, pl.ds(0, D)], sem_o)
    cc = pltpu.make_async_copy(
        col_s, out_ref.at[pl.ds(1, N), 0, pl.ds(D, D)], sem_o)
    ccls = pltpu.make_async_copy(cls_ref, out_ref.at[pl.ds(0, 1)], sem_o)
    cr.start()
    cc.start()
    ccls.start()

    cr.wait()
    cc.wait()
    ccls.wait()
    ct.wait()


def kernel(x, row_embed, col_embed, time_embed, cls_token_pos):
    out = pl.pallas_call(
        _pos_emb_kernel,
        in_specs=[
            pl.BlockSpec(memory_space=pltpu.MemorySpace.VMEM),
            pl.BlockSpec(memory_space=pltpu.MemorySpace.VMEM),
            pl.BlockSpec(memory_space=pltpu.MemorySpace.HBM),
            pl.BlockSpec(memory_space=pltpu.MemorySpace.HBM),
        ],
        out_specs=pl.BlockSpec(memory_space=pltpu.MemorySpace.HBM),
        out_shape=jax.ShapeDtypeStruct((N + 1, 1, EMBED_DIM), jnp.float32),
        scratch_shapes=[
            pltpu.VMEM((N, D), jnp.float32),
            pltpu.VMEM((N, D), jnp.float32),
            pltpu.SemaphoreType.DMA,
            pltpu.SemaphoreType.DMA,
        ],
    )(row_embed, col_embed, time_embed,
      cls_token_pos.reshape(1, 1, EMBED_DIM))
    return out.reshape(1, N + 1, EMBED_DIM)
